# baseline (speedup 1.0000x reference)
import functools
import jax
import jax.numpy as jnp
from jax import lax
from jax.experimental import pallas as pl
from jax.experimental.pallas import tpu as pltpu

CUTOFF = 6.0
_TN = 128
_PALIGN = 256


def _dense_body(n_atoms, pn, rt, zij, nij, w1, b1, w2, b2, cv, df, nadj):
    f32 = jnp.float32
    b = pl.program_id(0)
    i = pl.program_id(1)
    x = rt[0, 0]
    y = rt[0, 1]
    z = rt[0, 2]
    r2 = x * x + y * y + z * z
    tiny = r2 > 1e-12
    safe = jnp.where(tiny, r2, 1.0)
    r = jnp.sqrt(safe)
    mask = jnp.where(tiny & (r < CUTOFF), f32(1.0), f32(0.0))
    inv_r = mask / r
    inv_r2 = mask / safe
    nb = (zij[0] > 0).astype(f32)
    q0 = inv_r * nb
    q1 = x * inv_r2 * nb
    q2 = y * inv_r2 * nb
    q3 = z * inv_r2 * nb
    a = (q0[..., None] * w1[0] + q1[..., None] * w1[1]
         + q2[..., None] * w1[2] + q3[..., None] * w1[3] + b1[0])
    h = jnp.tanh(a)
    w2v = w2[0]
    rowid = lax.broadcasted_iota(jnp.int32, (x.shape[0],), 0) + i * x.shape[0]
    valid = (rowid < n_atoms).astype(f32)
    ei = (jnp.sum(jnp.sum(h * w2v, axis=2) * nb, axis=1) + b2[0, 0]) * valid
    g = (1.0 - h * h) * w2v
    u0 = jnp.sum(g * w1[0], axis=2) * nb
    u1 = jnp.sum(g * w1[1], axis=2) * nb
    u2 = jnp.sum(g * w1[2], axis=2) * nb
    u3 = jnp.sum(g * w1[3], axis=2) * nb
    inv_r3 = inv_r * inv_r2
    inv_r4 = inv_r2 * inv_r2
    s = u1 * x + u2 * y + u3 * z
    t0 = u0 * inv_r3 + 2.0 * s * inv_r4
    fx = t0 * x - u1 * inv_r2
    fy = t0 * y - u2 * inv_r2
    fz = t0 * z - u3 * inv_r2
    df[0, 0] = fx
    df[0, 1] = fy
    df[0, 2] = fz
    bl = lax.rem(b, 2)
    nij0 = nij[0]
    for c in range(3):
        nadj[0, c] = nij0 + (bl * 3 + c) * pn
    zero = jnp.zeros(fx.shape[:1], f32)
    cv[0, 0] = jnp.sum(fx, axis=1)
    cv[0, 1] = jnp.sum(fy, axis=1)
    cv[0, 2] = jnp.sum(fz, axis=1)
    cv[0, 3] = jnp.sum(x * fx, axis=1)
    cv[0, 4] = jnp.sum(x * fy, axis=1)
    cv[0, 5] = jnp.sum(x * fz, axis=1)
    cv[0, 6] = jnp.sum(y * fx, axis=1)
    cv[0, 7] = jnp.sum(y * fy, axis=1)
    cv[0, 8] = jnp.sum(y * fz, axis=1)
    cv[0, 9] = jnp.sum(z * fx, axis=1)
    cv[0, 10] = jnp.sum(z * fy, axis=1)
    cv[0, 11] = jnp.sum(z * fz, axis=1)
    cv[0, 12] = ei
    cv[0, 13] = zero
    cv[0, 14] = zero
    cv[0, 15] = zero


def _dense(rt, zij, nij, w1, b1, w2, b2, n_atoms):
    B, _, PN, M = rt.shape
    nblk = PN // _TN
    return pl.pallas_call(
        functools.partial(_dense_body, n_atoms, PN),
        grid=(B, nblk),
        in_specs=[
            pl.BlockSpec((1, 3, _TN, M), lambda b, i: (b, 0, i, 0)),
            pl.BlockSpec((1, _TN, M), lambda b, i: (b, i, 0)),
            pl.BlockSpec((1, _TN, M), lambda b, i: (b, i, 0)),
            pl.BlockSpec((4, 32), lambda b, i: (0, 0)),
            pl.BlockSpec((1, 32), lambda b, i: (0, 0)),
            pl.BlockSpec((1, 32), lambda b, i: (0, 0)),
            pl.BlockSpec((1, 1), lambda b, i: (0, 0)),
        ],
        out_specs=[
            pl.BlockSpec((1, 16, _TN), lambda b, i: (b, 0, i)),
            pl.BlockSpec((1, 3, _TN, M), lambda b, i: (b, 0, i, 0)),
            pl.BlockSpec((1, 3, _TN, M), lambda b, i: (b, 0, i, 0)),
        ],
        out_shape=[
            jax.ShapeDtypeStruct((B, 16, PN), jnp.float32),
            jax.ShapeDtypeStruct((B, 3, PN, M), jnp.float32),
            jax.ShapeDtypeStruct((B, 3, PN, M), jnp.int32),
        ],
    )(rt, zij, nij, w1, b1, w2, b2)


def _combine_body(cv, p, force, sums):
    for c in range(3):
        force[0, c] = cv[0, c] - p[0, c]
    sums[0, 0] = jnp.sum(cv[0], axis=1)


def _combine(cv, p):
    B, _, PN = cv.shape
    return pl.pallas_call(
        _combine_body,
        grid=(B,),
        in_specs=[
            pl.BlockSpec((1, 16, PN), lambda b: (b, 0, 0)),
            pl.BlockSpec((1, 3, PN), lambda b: (b, 0, 0)),
        ],
        out_specs=[
            pl.BlockSpec((1, 3, PN), lambda b: (b, 0, 0)),
            pl.BlockSpec((1, 1, 16), lambda b: (b, 0, 0)),
        ],
        out_shape=[
            jax.ShapeDtypeStruct((B, 3, PN), jnp.float32),
            jax.ShapeDtypeStruct((B, 1, 16), jnp.float32),
        ],
    )(cv, p)


def _scatter_placeholder(nadj, df, PN):
    B = nadj.shape[0]
    flat_idx = nadj.reshape(B, -1)
    flat_upd = df.reshape(B, -1)
    accs = []
    for half in range(B // 2):
        idx2 = flat_idx[2 * half:2 * half + 2].reshape(-1)
        upd2 = flat_upd[2 * half:2 * half + 2].reshape(-1)
        acc = jnp.zeros((6 * PN,), jnp.float32).at[idx2].add(upd2)
        accs.append(acc)
    return jnp.concatenate(accs).reshape(B, 3, PN)


def kernel(element_map, Zi, Nij, Zij, Rij, n_ghost, W1, b1, W2, b2):
    B, N, M = Nij.shape
    PN = ((N + _PALIGN - 1) // _PALIGN) * _PALIGN
    pad = PN - N
    rt = jnp.moveaxis(Rij, 3, 1)
    if pad:
        rt = jnp.pad(rt, ((0, 0), (0, 0), (0, pad), (0, 0)))
        Zijp = jnp.pad(Zij, ((0, 0), (0, pad), (0, 0)))
        Nijp = jnp.pad(Nij, ((0, 0), (0, pad), (0, 0)))
    else:
        Zijp, Nijp = Zij, Nij
    w1 = W1.astype(jnp.float32)
    b1r = b1.reshape(1, 32).astype(jnp.float32)
    w2r = W2.reshape(1, 32).astype(jnp.float32)
    b2r = b2.reshape(1, 1).astype(jnp.float32)
    cv, df, nadj = _dense(rt, Zijp, Nijp, w1, b1r, w2r, b2r, N)
    p = _scatter_placeholder(nadj, df, PN)
    force_pl, sums = _combine(cv, p)
    ghost = jnp.asarray(n_ghost, jnp.float32) - 0.0
    Force = jnp.moveaxis(force_pl[:, :, :N], 1, 2) + ghost
    Ei = cv[:, 12, :N].reshape(B, N, 1)
    virial = jnp.moveaxis(cv[:, 3:12, :N], 1, 2)
    Virial = sums[:, 0, 3:12]
    return sums[:, 0, 12:13], Ei, Force, Virial, virial


# speedup vs baseline: 2.0049x; 2.0049x over previous
import functools
import jax
import jax.numpy as jnp
from jax import lax
from jax.experimental import pallas as pl
from jax.experimental.pallas import tpu as pltpu
from jax.experimental.pallas import tpu_sc as plsc

CUTOFF = 6.0
_TN = 128
_PALIGN = 256


def _dense_body(n_atoms, pn, rt, zij, nij, w1, b1, w2, b2, cv, df, nadj):
    f32 = jnp.float32
    b = pl.program_id(0)
    i = pl.program_id(1)
    x = rt[0, 0]
    y = rt[0, 1]
    z = rt[0, 2]
    r2 = x * x + y * y + z * z
    tiny = r2 > 1e-12
    safe = jnp.where(tiny, r2, 1.0)
    r = jnp.sqrt(safe)
    mask = jnp.where(tiny & (r < CUTOFF), f32(1.0), f32(0.0))
    inv_r = mask / r
    inv_r2 = mask / safe
    nb = (zij[0] > 0).astype(f32)
    q0 = inv_r * nb
    q1 = x * inv_r2 * nb
    q2 = y * inv_r2 * nb
    q3 = z * inv_r2 * nb
    a = (q0[..., None] * w1[0] + q1[..., None] * w1[1]
         + q2[..., None] * w1[2] + q3[..., None] * w1[3] + b1[0])
    h = jnp.tanh(a)
    w2v = w2[0]
    rowid = lax.broadcasted_iota(jnp.int32, (x.shape[0],), 0) + i * x.shape[0]
    valid = (rowid < n_atoms).astype(f32)
    ei = (jnp.sum(jnp.sum(h * w2v, axis=2) * nb, axis=1) + b2[0, 0]) * valid
    g = (1.0 - h * h) * w2v
    u0 = jnp.sum(g * w1[0], axis=2) * nb
    u1 = jnp.sum(g * w1[1], axis=2) * nb
    u2 = jnp.sum(g * w1[2], axis=2) * nb
    u3 = jnp.sum(g * w1[3], axis=2) * nb
    inv_r3 = inv_r * inv_r2
    inv_r4 = inv_r2 * inv_r2
    s = u1 * x + u2 * y + u3 * z
    t0 = u0 * inv_r3 + 2.0 * s * inv_r4
    fx = t0 * x - u1 * inv_r2
    fy = t0 * y - u2 * inv_r2
    fz = t0 * z - u3 * inv_r2
    df[0, 0] = fx
    df[0, 1] = fy
    df[0, 2] = fz
    bl = lax.rem(b, 2)
    nij0 = nij[0]
    for c in range(3):
        nadj[0, c] = nij0 + (bl * 3 + c) * pn
    zero = jnp.zeros(fx.shape[:1], f32)
    cv[0, 0] = jnp.sum(fx, axis=1)
    cv[0, 1] = jnp.sum(fy, axis=1)
    cv[0, 2] = jnp.sum(fz, axis=1)
    cv[0, 3] = jnp.sum(x * fx, axis=1)
    cv[0, 4] = jnp.sum(x * fy, axis=1)
    cv[0, 5] = jnp.sum(x * fz, axis=1)
    cv[0, 6] = jnp.sum(y * fx, axis=1)
    cv[0, 7] = jnp.sum(y * fy, axis=1)
    cv[0, 8] = jnp.sum(y * fz, axis=1)
    cv[0, 9] = jnp.sum(z * fx, axis=1)
    cv[0, 10] = jnp.sum(z * fy, axis=1)
    cv[0, 11] = jnp.sum(z * fz, axis=1)
    cv[0, 12] = ei
    cv[0, 13] = zero
    cv[0, 14] = zero
    cv[0, 15] = zero


def _dense(rt, zij, nij, w1, b1, w2, b2, n_atoms):
    B, _, PN, M = rt.shape
    nblk = PN // _TN
    return pl.pallas_call(
        functools.partial(_dense_body, n_atoms, PN),
        grid=(B, nblk),
        in_specs=[
            pl.BlockSpec((1, 3, _TN, M), lambda b, i: (b, 0, i, 0)),
            pl.BlockSpec((1, _TN, M), lambda b, i: (b, i, 0)),
            pl.BlockSpec((1, _TN, M), lambda b, i: (b, i, 0)),
            pl.BlockSpec((4, 32), lambda b, i: (0, 0)),
            pl.BlockSpec((1, 32), lambda b, i: (0, 0)),
            pl.BlockSpec((1, 32), lambda b, i: (0, 0)),
            pl.BlockSpec((1, 1), lambda b, i: (0, 0)),
        ],
        out_specs=[
            pl.BlockSpec((1, 16, _TN), lambda b, i: (b, 0, i)),
            pl.BlockSpec((1, 3, _TN, M), lambda b, i: (b, 0, i, 0)),
            pl.BlockSpec((1, 3, _TN, M), lambda b, i: (b, 0, i, 0)),
        ],
        out_shape=[
            jax.ShapeDtypeStruct((B, 16, PN), jnp.float32),
            jax.ShapeDtypeStruct((B, 3, PN, M), jnp.float32),
            jax.ShapeDtypeStruct((B, 3, PN, M), jnp.int32),
        ],
    )(rt, zij, nij, w1, b1, w2, b2)


def _combine_body(cv, p, force, sums):
    for c in range(3):
        force[0, c] = cv[0, c] - p[0, c]
    sums[0, 0] = jnp.sum(cv[0], axis=1)


def _combine(cv, p):
    B, _, PN = cv.shape
    return pl.pallas_call(
        _combine_body,
        grid=(B,),
        in_specs=[
            pl.BlockSpec((1, 16, PN), lambda b: (b, 0, 0)),
            pl.BlockSpec((1, 3, PN), lambda b: (b, 0, 0)),
        ],
        out_specs=[
            pl.BlockSpec((1, 3, PN), lambda b: (b, 0, 0)),
            pl.BlockSpec((1, 1, 16), lambda b: (b, 0, 0)),
        ],
        out_shape=[
            jax.ShapeDtypeStruct((B, 3, PN), jnp.float32),
            jax.ShapeDtypeStruct((B, 1, 16), jnp.float32),
        ],
    )(cv, p)


def _sc_scatter(idx_flat, upd_flat, PN):
    ROWS = idx_flat.shape[0]
    ROWS_PER_SC = ROWS // 2
    RPW = ROWS_PER_SC // 16
    CH = 128
    NCH = RPW // CH
    ACC = 6 * PN
    ZCH = ACC // 16

    mesh = plsc.VectorSubcoreMesh(core_axis_name="c", subcore_axis_name="s")

    @functools.partial(
        pl.kernel,
        out_type=jax.ShapeDtypeStruct((2 * ACC,), jnp.float32),
        mesh=mesh,
        scratch_types=[
            pltpu.VMEM((CH, 128), jnp.int32),
            pltpu.VMEM((CH, 128), jnp.float32),
            pltpu.VMEM((ZCH,), jnp.float32),
            pltpu.VMEM_SHARED((ACC,), jnp.float32),
        ],
    )
    def scat(idx_hbm, upd_hbm, p_hbm, idx_v, upd_v, zbuf, acc_sh):
        cid = lax.axis_index("c")
        sid = lax.axis_index("s")

        def zb(i, carry):
            zbuf[pl.ds(i * 16, 16)] = jnp.zeros((16,), jnp.float32)
            return carry
        lax.fori_loop(0, ZCH // 16, zb, 0)
        pltpu.sync_copy(zbuf, acc_sh.at[pl.ds(sid * ZCH, ZCH)])
        plsc.subcore_barrier()

        base_row = cid * ROWS_PER_SC + sid * RPW

        def chunk(t, carry):
            r0 = base_row + t * CH
            pltpu.sync_copy(idx_hbm.at[pl.ds(r0, CH)], idx_v)
            pltpu.sync_copy(upd_hbm.at[pl.ds(r0, CH)], upd_v)

            def row(j, c2):
                pltpu.sync_copy(upd_v.at[j], acc_sh.at[idx_v.at[j]], add=True)
                return c2
            lax.fori_loop(0, CH, row, 0)
            return carry
        lax.fori_loop(0, NCH, chunk, 0)
        plsc.subcore_barrier()
        pltpu.sync_copy(acc_sh.at[pl.ds(sid * ZCH, ZCH)],
                        p_hbm.at[pl.ds(cid * ACC + sid * ZCH, ZCH)])

    return scat(idx_flat, upd_flat)


def kernel(element_map, Zi, Nij, Zij, Rij, n_ghost, W1, b1, W2, b2):
    B, N, M = Nij.shape
    PN = ((N + _PALIGN - 1) // _PALIGN) * _PALIGN
    pad = PN - N
    rt = jnp.moveaxis(Rij, 3, 1)
    if pad:
        rt = jnp.pad(rt, ((0, 0), (0, 0), (0, pad), (0, 0)))
        Zijp = jnp.pad(Zij, ((0, 0), (0, pad), (0, 0)))
        Nijp = jnp.pad(Nij, ((0, 0), (0, pad), (0, 0)))
    else:
        Zijp, Nijp = Zij, Nij
    w1 = W1.astype(jnp.float32)
    b1r = b1.reshape(1, 32).astype(jnp.float32)
    w2r = W2.reshape(1, 32).astype(jnp.float32)
    b2r = b2.reshape(1, 1).astype(jnp.float32)
    cv, df, nadj = _dense(rt, Zijp, Nijp, w1, b1r, w2r, b2r, N)
    rows = B * 3 * PN * M // 128
    p_flat = _sc_scatter(nadj.reshape(rows, 128), df.reshape(rows, 128), PN)
    p = p_flat.reshape(B, 3, PN)
    force_pl, sums = _combine(cv, p)
    ghost = jnp.asarray(n_ghost, jnp.float32) - 0.0
    Force = jnp.moveaxis(force_pl[:, :, :N], 1, 2) + ghost
    Ei = cv[:, 12, :N].reshape(B, N, 1)
    virial = jnp.moveaxis(cv[:, 3:12, :N], 1, 2)
    Virial = sums[:, 0, 3:12]
    return sums[:, 0, 12:13], Ei, Force, Virial, virial


# speedup vs baseline: 13.2437x; 6.6056x over previous
import functools
import jax
import jax.numpy as jnp
from jax import lax
from jax.experimental import pallas as pl
from jax.experimental.pallas import tpu as pltpu
from jax.experimental.pallas import tpu_sc as plsc

CUTOFF = 6.0
_TN = 128
_PALIGN = 256


def _dense_body(n_atoms, pn, rt, zij, nij, w1, b1, w2, b2, cv, df, nadj):
    f32 = jnp.float32
    b = pl.program_id(0)
    i = pl.program_id(1)
    G, M = rt.shape[2], 64

    def seg_sum(arr):
        s0 = jnp.sum(arr[:, :M], axis=1)
        s1 = jnp.sum(arr[:, M:], axis=1)
        return jnp.stack([s0, s1], axis=1)

    x = rt[0, 0]
    y = rt[0, 1]
    z = rt[0, 2]
    r2 = x * x + y * y + z * z
    tiny = r2 > 1e-12
    safe = jnp.where(tiny, r2, 1.0)
    r = jnp.sqrt(safe)
    mask = jnp.where(tiny & (r < CUTOFF), f32(1.0), f32(0.0))
    inv_r = mask / r
    inv_r2 = mask / safe
    nb = (zij[0] > 0).astype(f32)
    q0 = inv_r * nb
    q1 = x * inv_r2 * nb
    q2 = y * inv_r2 * nb
    q3 = z * inv_r2 * nb
    w1c = [w1[k][:, None] for k in range(4)]
    a3 = (q0[:, None, :] * w1c[0] + q1[:, None, :] * w1c[1]
          + q2[:, None, :] * w1c[2] + q3[:, None, :] * w1c[3]
          + b1[0][:, None])
    h3 = jnp.tanh(a3)
    w2c = w2[0][:, None]
    atom = (i * (2 * G) + 2 * lax.broadcasted_iota(jnp.int32, (G, 2), 0)
            + lax.broadcasted_iota(jnp.int32, (G, 2), 1))
    valid = (atom < n_atoms).astype(f32)
    ei_pair = jnp.sum(h3 * w2c, axis=1) * nb
    ei = (seg_sum(ei_pair) + b2[0, 0]) * valid
    g3 = (1.0 - h3 * h3) * w2c
    u0 = jnp.sum(g3 * w1c[0], axis=1) * nb
    u1 = jnp.sum(g3 * w1c[1], axis=1) * nb
    u2 = jnp.sum(g3 * w1c[2], axis=1) * nb
    u3 = jnp.sum(g3 * w1c[3], axis=1) * nb
    inv_r3 = inv_r * inv_r2
    inv_r4 = inv_r2 * inv_r2
    s = u1 * x + u2 * y + u3 * z
    t0 = u0 * inv_r3 + 2.0 * s * inv_r4
    fx = t0 * x - u1 * inv_r2
    fy = t0 * y - u2 * inv_r2
    fz = t0 * z - u3 * inv_r2
    df[0, 0] = fx
    df[0, 1] = fy
    df[0, 2] = fz
    bl = lax.rem(b, 2)
    nij0 = nij[0]
    for c in range(3):
        nadj[0, c] = nij0 + (bl * 3 + c) * pn
    zero = jnp.zeros((G, 2), f32)
    cv[0, 0] = seg_sum(fx)
    cv[0, 1] = seg_sum(fy)
    cv[0, 2] = seg_sum(fz)
    cv[0, 3] = seg_sum(x * fx)
    cv[0, 4] = seg_sum(x * fy)
    cv[0, 5] = seg_sum(x * fz)
    cv[0, 6] = seg_sum(y * fx)
    cv[0, 7] = seg_sum(y * fy)
    cv[0, 8] = seg_sum(y * fz)
    cv[0, 9] = seg_sum(z * fx)
    cv[0, 10] = seg_sum(z * fy)
    cv[0, 11] = seg_sum(z * fz)
    cv[0, 12] = ei
    cv[0, 13] = zero
    cv[0, 14] = zero
    cv[0, 15] = zero


def _dense(rtf, zijf, nijf, w1, b1, w2, b2, n_atoms, PN):
    B, _, GT, _ = rtf.shape
    G = _TN * 64 // 128
    nblk = GT // G
    return pl.pallas_call(
        functools.partial(_dense_body, n_atoms, PN),
        grid=(B, nblk),
        in_specs=[
            pl.BlockSpec((1, 3, G, 128), lambda b, i: (b, 0, i, 0)),
            pl.BlockSpec((1, G, 128), lambda b, i: (b, i, 0)),
            pl.BlockSpec((1, G, 128), lambda b, i: (b, i, 0)),
            pl.BlockSpec((4, 32), lambda b, i: (0, 0)),
            pl.BlockSpec((1, 32), lambda b, i: (0, 0)),
            pl.BlockSpec((1, 32), lambda b, i: (0, 0)),
            pl.BlockSpec((1, 1), lambda b, i: (0, 0)),
        ],
        out_specs=[
            pl.BlockSpec((1, 16, G, 2), lambda b, i: (b, 0, i, 0)),
            pl.BlockSpec((1, 3, G, 128), lambda b, i: (b, 0, i, 0)),
            pl.BlockSpec((1, 3, G, 128), lambda b, i: (b, 0, i, 0)),
        ],
        out_shape=[
            jax.ShapeDtypeStruct((B, 16, GT, 2), jnp.float32),
            jax.ShapeDtypeStruct((B, 3, GT, 128), jnp.float32),
            jax.ShapeDtypeStruct((B, 3, GT, 128), jnp.int32),
        ],
    )(rtf, zijf, nijf, w1, b1, w2, b2)


def _combine_body(cv, p, force, sums):
    for c in range(3):
        force[0, c] = cv[0, c] - p[0, c]
    sums[0, 0] = jnp.sum(cv[0], axis=1)


def _combine(cv, p):
    B, _, PN = cv.shape
    return pl.pallas_call(
        _combine_body,
        grid=(B,),
        in_specs=[
            pl.BlockSpec((1, 16, PN), lambda b: (b, 0, 0)),
            pl.BlockSpec((1, 3, PN), lambda b: (b, 0, 0)),
        ],
        out_specs=[
            pl.BlockSpec((1, 3, PN), lambda b: (b, 0, 0)),
            pl.BlockSpec((1, 1, 16), lambda b: (b, 0, 0)),
        ],
        out_shape=[
            jax.ShapeDtypeStruct((B, 3, PN), jnp.float32),
            jax.ShapeDtypeStruct((B, 1, 16), jnp.float32),
        ],
    )(cv, p)


def _sc_scatter(idx_flat, upd_flat, PN):
    ROWS = idx_flat.shape[0]
    ROWS_PER_SC = ROWS // 2
    RPW = ROWS_PER_SC // 16
    CH = 128
    NCH = RPW // CH
    ACC = 6 * PN
    ZCH = ACC // 16

    mesh = plsc.VectorSubcoreMesh(core_axis_name="c", subcore_axis_name="s")

    @functools.partial(
        pl.kernel,
        out_type=jax.ShapeDtypeStruct((2 * ACC,), jnp.float32),
        mesh=mesh,
        scratch_types=[
            pltpu.VMEM((CH, 128), jnp.int32),
            pltpu.VMEM((CH, 128), jnp.float32),
            pltpu.VMEM((ZCH,), jnp.float32),
            pltpu.VMEM_SHARED((ACC,), jnp.float32),
        ],
    )
    def scat(idx_hbm, upd_hbm, p_hbm, idx_v, upd_v, zbuf, acc_sh):
        cid = lax.axis_index("c")
        sid = lax.axis_index("s")

        def zb(i, carry):
            zbuf[pl.ds(i * 16, 16)] = jnp.zeros((16,), jnp.float32)
            return carry
        lax.fori_loop(0, ZCH // 16, zb, 0)
        pltpu.sync_copy(zbuf, acc_sh.at[pl.ds(sid * ZCH, ZCH)])
        plsc.subcore_barrier()

        base_row = cid * ROWS_PER_SC + sid * RPW

        def chunk(t, carry):
            r0 = base_row + t * CH
            pltpu.sync_copy(idx_hbm.at[pl.ds(r0, CH)], idx_v)
            pltpu.sync_copy(upd_hbm.at[pl.ds(r0, CH)], upd_v)

            def row(j, c2):
                pltpu.sync_copy(upd_v.at[j], acc_sh.at[idx_v.at[j]], add=True)
                return c2
            lax.fori_loop(0, CH, row, 0)
            return carry
        lax.fori_loop(0, NCH, chunk, 0)
        plsc.subcore_barrier()
        pltpu.sync_copy(acc_sh.at[pl.ds(sid * ZCH, ZCH)],
                        p_hbm.at[pl.ds(cid * ACC + sid * ZCH, ZCH)])

    return scat(idx_flat, upd_flat)


def kernel(element_map, Zi, Nij, Zij, Rij, n_ghost, W1, b1, W2, b2):
    B, N, M = Nij.shape
    PN = ((N + _PALIGN - 1) // _PALIGN) * _PALIGN
    pad = PN - N
    rt = jnp.moveaxis(Rij, 3, 1)
    if pad:
        rt = jnp.pad(rt, ((0, 0), (0, 0), (0, pad), (0, 0)))
        Zijp = jnp.pad(Zij, ((0, 0), (0, pad), (0, 0)))
        Nijp = jnp.pad(Nij, ((0, 0), (0, pad), (0, 0)))
    else:
        Zijp, Nijp = Zij, Nij
    w1 = W1.astype(jnp.float32)
    b1r = b1.reshape(1, 32).astype(jnp.float32)
    w2r = W2.reshape(1, 32).astype(jnp.float32)
    b2r = b2.reshape(1, 1).astype(jnp.float32)
    GT = PN * M // 128
    rtf = rt.reshape(B, 3, GT, 128)
    zijf = Zijp.reshape(B, GT, 128)
    nijf = Nijp.reshape(B, GT, 128)
    cv4, df, nadj = _dense(rtf, zijf, nijf, w1, b1r, w2r, b2r, N, PN)
    cv = cv4.reshape(B, 16, PN)
    rows = B * 3 * GT
    p_flat = _sc_scatter(nadj.reshape(rows, 128), df.reshape(rows, 128), PN)
    p = p_flat.reshape(B, 3, PN)
    force_pl, sums = _combine(cv, p)
    ghost = jnp.asarray(n_ghost, jnp.float32) - 0.0
    Force = jnp.moveaxis(force_pl[:, :, :N], 1, 2) + ghost
    Ei = cv[:, 12, :N].reshape(B, N, 1)
    virial = jnp.moveaxis(cv[:, 3:12, :N], 1, 2)
    Virial = sums[:, 0, 3:12]
    return sums[:, 0, 12:13], Ei, Force, Virial, virial
